# baseline (speedup 1.0000x reference)
import functools

import jax
import jax.numpy as jnp
from jax.experimental import pallas as pl
from jax.experimental.pallas import tpu as pltpu

_LOG2E = 1.4426950408889634


def _proj_kernel(h_ref, w_ref, A_ref, g_ref, el_ref, ert_ref, *, n_heads):
    g = jnp.dot(h_ref[...], w_ref[...], preferred_element_type=jnp.float32)
    g_ref[...] = g
    elr = jnp.dot(g, A_ref[...], preferred_element_type=jnp.float32)
    el_ref[...] = elr[:, :n_heads]
    ert_ref[...] = elr[:, n_heads:].T


def _attn_kernel(adj_ref, g_ref, el_ref, ert_ref, o_ref, *, n_heads,
                 n_hidden, apply_elu):
    neg = jnp.where(adj_ref[...], 0.0, -jnp.inf)
    acc = None
    for hd in range(n_heads):
        e0 = el_ref[:, hd:hd + 1] + ert_ref[hd:hd + 1, :]
        t = jnp.maximum(e0, 0.2 * e0) + neg
        m = jnp.max(t, axis=1, keepdims=True)
        p = jnp.exp2(t - m)
        s = jnp.sum(p, axis=1, keepdims=True)
        gh = g_ref[:, hd * n_hidden:(hd + 1) * n_hidden]
        o_h = jnp.dot(p, gh, preferred_element_type=jnp.float32) / s
        acc = o_h if acc is None else acc + o_h
    out = acc * (1.0 / n_heads)
    if apply_elu:
        out = jnp.where(out > 0, out, jnp.exp(out) - 1.0)
    o_ref[...] = out


def _build_A(a, n_heads, n_hidden):
    a_l = a[:n_hidden]
    a_r = a[n_hidden:]
    A = jnp.zeros((n_heads * n_hidden, 2 * n_heads), jnp.float32)
    for h in range(n_heads):
        A = A.at[h * n_hidden:(h + 1) * n_hidden, h].set(a_l)
        A = A.at[h * n_hidden:(h + 1) * n_hidden, n_heads + h].set(a_r)
    return A * _LOG2E


def _gat_layer(h, adj, W, a, n_heads, n_hidden, apply_elu, block_m=256):
    n = h.shape[0]
    A = _build_A(a, n_heads, n_hidden)
    g, el, ert = pl.pallas_call(
        functools.partial(_proj_kernel, n_heads=n_heads),
        out_shape=[
            jax.ShapeDtypeStruct((n, n_heads * n_hidden), jnp.float32),
            jax.ShapeDtypeStruct((n, n_heads), jnp.float32),
            jax.ShapeDtypeStruct((n_heads, n), jnp.float32),
        ],
    )(h, W, A)

    return pl.pallas_call(
        functools.partial(_attn_kernel, n_heads=n_heads, n_hidden=n_hidden,
                          apply_elu=apply_elu),
        grid=(n // block_m,),
        in_specs=[
            pl.BlockSpec((block_m, n), lambda i: (i, 0)),
            pl.BlockSpec((n, n_heads * n_hidden), lambda i: (0, 0)),
            pl.BlockSpec((block_m, n_heads), lambda i: (i, 0)),
            pl.BlockSpec((n_heads, n), lambda i: (0, 0)),
        ],
        out_specs=pl.BlockSpec((block_m, n_hidden), lambda i: (i, 0)),
        out_shape=jax.ShapeDtypeStruct((n, n_hidden), jnp.float32),
        compiler_params=pltpu.CompilerParams(
            dimension_semantics=("parallel",)),
    )(adj, g, el, ert)


def kernel(x, adj_mat, W1, a1, W2, a2):
    n = x.shape[0]
    n_hidden = a1.shape[0] // 2
    n_heads = W1.shape[1] // n_hidden
    n_classes = a2.shape[0] // 2
    adj = adj_mat.reshape(n, n)
    h1 = _gat_layer(x, adj, W1, a1, n_heads, n_hidden, apply_elu=True)
    return _gat_layer(h1, adj, W2, a2, 1, n_classes, apply_elu=False)


# speedup vs baseline: 1.0985x; 1.0985x over previous
import functools

import jax
import jax.numpy as jnp
from jax.experimental import pallas as pl
from jax.experimental.pallas import tpu as pltpu

_LOG2E = 1.4426950408889634


def _layer_kernel(h_ref, w_ref, A_ref, adj_ref, o_ref, g_ref, elr_ref,
                  ert_ref, *, n_heads, n_hidden, block_m, apply_elu):
    i = pl.program_id(0)

    @pl.when(i == 0)
    def _():
        g = jnp.dot(h_ref[...], w_ref[...], preferred_element_type=jnp.float32)
        g_ref[...] = g
        elr = jnp.dot(g, A_ref[...], preferred_element_type=jnp.float32)
        elr_ref[...] = elr
        ert_ref[...] = elr[:, n_heads:].T

    neg = jnp.where(adj_ref[...], 0.0, -jnp.inf)
    acc = None
    for hd in range(n_heads):
        el_h = elr_ref[pl.ds(i * block_m, block_m), hd:hd + 1]
        e0 = el_h + ert_ref[hd:hd + 1, :]
        t = jnp.maximum(e0, 0.2 * e0) + neg
        m = jnp.max(t, axis=1, keepdims=True)
        p = jnp.exp2(t - m)
        s = jnp.sum(p, axis=1, keepdims=True)
        gh = g_ref[:, hd * n_hidden:(hd + 1) * n_hidden]
        o_h = jnp.dot(p, gh, preferred_element_type=jnp.float32) / s
        acc = o_h if acc is None else acc + o_h
    out = acc * (1.0 / n_heads)
    if apply_elu:
        out = jnp.where(out > 0, out, jnp.exp(out) - 1.0)
    o_ref[...] = out


def _build_A(a, n_heads, n_hidden):
    a_l = a[:n_hidden]
    a_r = a[n_hidden:]
    A = jnp.zeros((n_heads * n_hidden, 2 * n_heads), jnp.float32)
    for h in range(n_heads):
        A = A.at[h * n_hidden:(h + 1) * n_hidden, h].set(a_l)
        A = A.at[h * n_hidden:(h + 1) * n_hidden, n_heads + h].set(a_r)
    return A * _LOG2E


def _gat_layer(h, adj, W, a, n_heads, n_hidden, apply_elu, block_m=512):
    n = h.shape[0]
    A = _build_A(a, n_heads, n_hidden)
    return pl.pallas_call(
        functools.partial(_layer_kernel, n_heads=n_heads, n_hidden=n_hidden,
                          block_m=block_m, apply_elu=apply_elu),
        grid=(n // block_m,),
        in_specs=[
            pl.BlockSpec((n, h.shape[1]), lambda i: (0, 0)),
            pl.BlockSpec(W.shape, lambda i: (0, 0)),
            pl.BlockSpec(A.shape, lambda i: (0, 0)),
            pl.BlockSpec((block_m, n), lambda i: (i, 0)),
        ],
        out_specs=pl.BlockSpec((block_m, n_hidden), lambda i: (i, 0)),
        out_shape=jax.ShapeDtypeStruct((n, n_hidden), jnp.float32),
        scratch_shapes=[
            pltpu.VMEM((n, n_heads * n_hidden), jnp.float32),
            pltpu.VMEM((n, 2 * n_heads), jnp.float32),
            pltpu.VMEM((n_heads, n), jnp.float32),
        ],
        compiler_params=pltpu.CompilerParams(
            dimension_semantics=("arbitrary",)),
    )(h, W, A, adj)


def kernel(x, adj_mat, W1, a1, W2, a2):
    n = x.shape[0]
    n_hidden = a1.shape[0] // 2
    n_heads = W1.shape[1] // n_hidden
    n_classes = a2.shape[0] // 2
    adj = adj_mat.reshape(n, n)
    h1 = _gat_layer(x, adj, W1, a1, n_heads, n_hidden, apply_elu=True)
    return _gat_layer(h1, adj, W2, a2, 1, n_classes, apply_elu=False)


# speedup vs baseline: 1.3610x; 1.2390x over previous
import functools

import jax
import jax.numpy as jnp
from jax.experimental import pallas as pl
from jax.experimental.pallas import tpu as pltpu

_LOG2E = 1.4426950408889634
_SLOT = 128


def _layer_kernel(h_ref, w_ref, A_ref, adj_ref, o_ref, gx_ref, elr_ref,
                  ert_ref, mx_ref, *, n_heads, n_hidden, block_m, apply_elu):
    i = pl.program_id(0)

    @pl.when(i == 0)
    def _():
        g = jnp.dot(h_ref[...], w_ref[...], preferred_element_type=jnp.float32)
        elr = jnp.dot(g, A_ref[...], preferred_element_type=jnp.float32)
        elr_ref[...] = elr
        ert = elr[:, n_heads:].T
        ert_ref[...] = ert
        mx_ref[...] = jnp.max(ert, axis=1, keepdims=True)
        n = g.shape[0]
        for hd in range(n_heads):
            base = hd * _SLOT
            gx_ref[:, base:base + n_hidden] = \
                g[:, hd * n_hidden:(hd + 1) * n_hidden]
            gx_ref[:, base + n_hidden:base + n_hidden + 1] = \
                jnp.ones((n, 1), jnp.float32)

    use_neg = n_heads > 1
    if use_neg:
        neg = jnp.where(adj_ref[...], 0.0, -jnp.inf)
    acc = None
    for hd in range(n_heads):
        el_h = elr_ref[pl.ds(i * block_m, block_m), hd:hd + 1]
        mu = el_h + mx_ref[hd:hd + 1, 0:1]
        mub = jnp.maximum(mu, 0.2 * mu)
        c1 = el_h - mub
        c2 = 0.2 * el_h - mub
        er_row = ert_ref[hd:hd + 1, :]
        t = jnp.maximum(c1 + er_row, c2 + 0.2 * er_row)
        if use_neg:
            p = jnp.exp2(t + neg)
        else:
            p = jnp.where(adj_ref[...], jnp.exp2(t), 0.0)
        ox = jnp.dot(p, gx_ref[:, hd * _SLOT:(hd + 1) * _SLOT],
                     preferred_element_type=jnp.float32)
        o_h = ox[:, :n_hidden] / ox[:, n_hidden:n_hidden + 1]
        acc = o_h if acc is None else acc + o_h
    out = acc * (1.0 / n_heads)
    if apply_elu:
        out = jnp.where(out > 0, out, jnp.exp(out) - 1.0)
    o_ref[...] = out


def _build_A(a, n_heads, n_hidden):
    a_l = a[:n_hidden]
    a_r = a[n_hidden:]
    A = jnp.zeros((n_heads * n_hidden, 2 * n_heads), jnp.float32)
    for h in range(n_heads):
        A = A.at[h * n_hidden:(h + 1) * n_hidden, h].set(a_l)
        A = A.at[h * n_hidden:(h + 1) * n_hidden, n_heads + h].set(a_r)
    return A * _LOG2E


def _gat_layer(h, adj, W, a, n_heads, n_hidden, apply_elu, block_m=512):
    n = h.shape[0]
    A = _build_A(a, n_heads, n_hidden)
    return pl.pallas_call(
        functools.partial(_layer_kernel, n_heads=n_heads, n_hidden=n_hidden,
                          block_m=block_m, apply_elu=apply_elu),
        grid=(n // block_m,),
        in_specs=[
            pl.BlockSpec((n, h.shape[1]), lambda i: (0, 0)),
            pl.BlockSpec(W.shape, lambda i: (0, 0)),
            pl.BlockSpec(A.shape, lambda i: (0, 0)),
            pl.BlockSpec((block_m, n), lambda i: (i, 0)),
        ],
        out_specs=pl.BlockSpec((block_m, n_hidden), lambda i: (i, 0)),
        out_shape=jax.ShapeDtypeStruct((n, n_hidden), jnp.float32),
        scratch_shapes=[
            pltpu.VMEM((n, n_heads * _SLOT), jnp.float32),
            pltpu.VMEM((n, 2 * n_heads), jnp.float32),
            pltpu.VMEM((n_heads, n), jnp.float32),
            pltpu.VMEM((n_heads, 1), jnp.float32),
        ],
        compiler_params=pltpu.CompilerParams(
            dimension_semantics=("arbitrary",)),
    )(h, W, A, adj)


def kernel(x, adj_mat, W1, a1, W2, a2):
    n = x.shape[0]
    n_hidden = a1.shape[0] // 2
    n_heads = W1.shape[1] // n_hidden
    n_classes = a2.shape[0] // 2
    adj = adj_mat.reshape(n, n)
    h1 = _gat_layer(x, adj, W1, a1, n_heads, n_hidden, apply_elu=True)
    return _gat_layer(h1, adj, W2, a2, 1, n_classes, apply_elu=False)


# speedup vs baseline: 1.3838x; 1.0167x over previous
import functools

import jax
import jax.numpy as jnp
from jax.experimental import pallas as pl
from jax.experimental.pallas import tpu as pltpu

_LOG2E = 1.4426950408889634
_SLOT = 128


def _layer_kernel(h_ref, w_ref, A_ref, adj_ref, o_ref, gx_ref, elr_ref,
                  ert_ref, mx_ref, *, n_heads, n_hidden, block_m, apply_elu):
    i = pl.program_id(0)

    @pl.when(i == 0)
    def _():
        g = jnp.dot(h_ref[...], w_ref[...], preferred_element_type=jnp.float32)
        elr = jnp.dot(g, A_ref[...], preferred_element_type=jnp.float32)
        elr_ref[...] = elr
        ert = elr[:, n_heads:].T
        ert_ref[...] = ert
        mx_ref[...] = jnp.max(ert, axis=1, keepdims=True)
        n = g.shape[0]
        for hd in range(n_heads):
            base = hd * _SLOT
            gx_ref[:, base:base + n_hidden] = \
                g[:, hd * n_hidden:(hd + 1) * n_hidden].astype(jnp.bfloat16)
            gx_ref[:, base + n_hidden:base + n_hidden + 1] = \
                jnp.ones((n, 1), jnp.bfloat16)

    adj = adj_ref[...]
    acc = None
    for hd in range(n_heads):
        el_h = elr_ref[pl.ds(i * block_m, block_m), hd:hd + 1]
        mu = el_h + mx_ref[hd:hd + 1, 0:1]
        mub = jnp.maximum(mu, 0.2 * mu)
        c1 = el_h - mub
        c2 = 0.2 * el_h - mub
        er_row = ert_ref[hd:hd + 1, :]
        t = jnp.maximum(c1 + er_row, c2 + 0.2 * er_row)
        p = jnp.where(adj, jnp.exp2(t), 0.0).astype(jnp.bfloat16)
        ox = jnp.dot(p, gx_ref[:, hd * _SLOT:(hd + 1) * _SLOT],
                     preferred_element_type=jnp.float32)
        o_h = ox[:, :n_hidden] / ox[:, n_hidden:n_hidden + 1]
        acc = o_h if acc is None else acc + o_h
    out = acc * (1.0 / n_heads)
    if apply_elu:
        out = jnp.where(out > 0, out, jnp.exp(out) - 1.0)
    o_ref[...] = out


def _build_A(a, n_heads, n_hidden):
    a_l = a[:n_hidden]
    a_r = a[n_hidden:]
    A = jnp.zeros((n_heads * n_hidden, 2 * n_heads), jnp.float32)
    for h in range(n_heads):
        A = A.at[h * n_hidden:(h + 1) * n_hidden, h].set(a_l)
        A = A.at[h * n_hidden:(h + 1) * n_hidden, n_heads + h].set(a_r)
    return A * _LOG2E


def _gat_layer(h, adj, W, a, n_heads, n_hidden, apply_elu, block_m=512):
    n = h.shape[0]
    A = _build_A(a, n_heads, n_hidden)
    return pl.pallas_call(
        functools.partial(_layer_kernel, n_heads=n_heads, n_hidden=n_hidden,
                          block_m=block_m, apply_elu=apply_elu),
        grid=(n // block_m,),
        in_specs=[
            pl.BlockSpec((n, h.shape[1]), lambda i: (0, 0)),
            pl.BlockSpec(W.shape, lambda i: (0, 0)),
            pl.BlockSpec(A.shape, lambda i: (0, 0)),
            pl.BlockSpec((block_m, n), lambda i: (i, 0)),
        ],
        out_specs=pl.BlockSpec((block_m, n_hidden), lambda i: (i, 0)),
        out_shape=jax.ShapeDtypeStruct((n, n_hidden), jnp.float32),
        scratch_shapes=[
            pltpu.VMEM((n, n_heads * _SLOT), jnp.bfloat16),
            pltpu.VMEM((n, 2 * n_heads), jnp.float32),
            pltpu.VMEM((n_heads, n), jnp.float32),
            pltpu.VMEM((n_heads, 1), jnp.float32),
        ],
        compiler_params=pltpu.CompilerParams(
            dimension_semantics=("arbitrary",)),
    )(h, W, A, adj)


def kernel(x, adj_mat, W1, a1, W2, a2):
    n = x.shape[0]
    n_hidden = a1.shape[0] // 2
    n_heads = W1.shape[1] // n_hidden
    n_classes = a2.shape[0] // 2
    adj = adj_mat.reshape(n, n)
    h1 = _gat_layer(x, adj, W1, a1, n_heads, n_hidden, apply_elu=True)
    return _gat_layer(h1, adj, W2, a2, 1, n_classes, apply_elu=False)
